# baseline (speedup 1.0000x reference)
import jax
import jax.numpy as jnp
from jax import lax
from jax.experimental import pallas as pl
from jax.experimental.pallas import tpu as pltpu
from jax.experimental.pallas import tpu_sc as plsc

A = 589824
EPS = 1e-7
NC = 2
NS = 16
NW = NC * NS

TOTAL_ROWS = A * 4 // 128
LBL_ROWS = A // 128

SC_A = 147456
SC_PER_W = SC_A // NW
CHUNK = SC_PER_W // 4
NCHUNK = SC_PER_W // CHUNK
GROUPS = CHUNK // 16

SC_ROWS = SC_A * 4 // 128
BR = 512
BL = BR // 4
TC_GRID = (TOTAL_ROWS - SC_ROWS) // BR
TC_ROW_OFF = SC_ROWS // BR
TC_LBL_OFF = (SC_A // 128) // BL


def _sc_body(out_hbm, tgt_hbm, lbl_hbm, res_hbm,
             obuf0, tbuf0, lbuf0, obuf1, tbuf1, lbuf1, res_v,
             sem0, sem1):
    wid = lax.axis_index("s") * NC + lax.axis_index("c")
    a_base = wid * SC_PER_W
    bufs = ((obuf0, tbuf0, lbuf0, sem0), (obuf1, tbuf1, lbuf1, sem1))

    def issue(c):
        ob, tb, lb, sem = bufs[c % 2]
        astart = a_base + c * CHUNK
        return (
            pltpu.async_copy(out_hbm.at[pl.ds(astart * 4, CHUNK * 4)], ob, sem),
            pltpu.async_copy(tgt_hbm.at[pl.ds(astart * 4, CHUNK * 4)], tb, sem),
            pltpu.async_copy(lbl_hbm.at[pl.ds(astart, CHUNK)], lb, sem),
        )

    pending = [issue(0), issue(1)]

    zero = jnp.zeros((16,), jnp.float32)
    one = jnp.ones((16,), jnp.float32)
    a_acc, p_acc, v_acc = zero, zero, zero

    for c in range(NCHUNK):
        for d in pending[c]:
            d.wait()
        ob, tb, lb, _ = bufs[c % 2]

        def jbody(j, carry, ob=ob, tb=tb, lb=lb):
            aa, pa, va = carry
            base = ((j >> 3) << 9) + ((j & 7) << 4)
            lbl16 = lb[pl.ds(j * 16, 16)]
            m16 = jnp.where(lbl16 > 0.0, one, zero)
            pa = pa + m16
            va = va + jnp.where(lbl16 != -1.0, one, zero)
            s = zero
            for k in range(4):
                o = ob[pl.ds(base + k * 128, 16)]
                t = tb[pl.ds(base + k * 128, 16)]
                diff = jnp.abs(o - t)
                md = jnp.minimum(diff, 1.0)
                s = s + md * (diff + diff - md)
            aa = aa + m16 * s
            return aa, pa, va

        a_acc, p_acc, v_acc = plsc.parallel_loop(
            0, GROUPS, unroll=2, carry=(a_acc, p_acc, v_acc))(jbody)

        nxt = c + 2
        if nxt < NCHUNK:
            pending.append(issue(nxt))

    res_v[pl.ds(0, 16)] = a_acc
    res_v[pl.ds(16, 16)] = p_acc
    res_v[pl.ds(32, 16)] = v_acc
    pltpu.sync_copy(res_v, res_hbm.at[wid])


def _tc_body(dref, tref, lref, oref):
    i = pl.program_id(0)

    @pl.when(i == 0)
    def _():
        oref[...] = jnp.zeros_like(oref)

    lbl = lref[...]
    m = jnp.where(lbl > 0.0, 1.0, 0.0)
    mexp = jnp.reshape(
        jnp.broadcast_to(m[:, None, :], (BL, 4, 128)), (BR, 128))
    d = jnp.abs(dref[...] - tref[...])
    md = jnp.minimum(d, 1.0)
    s2 = md * (d + d - md)
    arow = jnp.sum(mexp * s2, axis=0)
    prow = jnp.sum(m, axis=0)
    vrow = jnp.sum(jnp.where(lbl != -1.0, 1.0, 0.0), axis=0)
    oref[0, :] += arow
    oref[1, :] += prow
    oref[2, :] += vrow


@jax.jit
def _rpn_loss(out_flat, tgt_flat, lbl_flat):
    mesh = plsc.VectorSubcoreMesh(core_axis_name="c", subcore_axis_name="s")
    sc_partials = pl.kernel(
        _sc_body,
        mesh=mesh,
        out_type=jax.ShapeDtypeStruct((NW, 48), jnp.float32),
        scratch_types=[
            pltpu.VMEM((CHUNK * 4,), jnp.float32),
            pltpu.VMEM((CHUNK * 4,), jnp.float32),
            pltpu.VMEM((CHUNK,), jnp.float32),
            pltpu.VMEM((CHUNK * 4,), jnp.float32),
            pltpu.VMEM((CHUNK * 4,), jnp.float32),
            pltpu.VMEM((CHUNK,), jnp.float32),
            pltpu.VMEM((48,), jnp.float32),
            pltpu.SemaphoreType.DMA,
            pltpu.SemaphoreType.DMA,
        ],
    )(out_flat, tgt_flat, lbl_flat)

    d2 = jnp.reshape(out_flat, (TOTAL_ROWS, 128))
    t2 = jnp.reshape(tgt_flat, (TOTAL_ROWS, 128))
    l2 = jnp.reshape(lbl_flat, (LBL_ROWS, 128))
    tc_partials = pl.pallas_call(
        _tc_body,
        grid=(TC_GRID,),
        in_specs=[
            pl.BlockSpec((BR, 128), lambda i: (TC_ROW_OFF + i, 0)),
            pl.BlockSpec((BR, 128), lambda i: (TC_ROW_OFF + i, 0)),
            pl.BlockSpec((BL, 128), lambda i: (TC_LBL_OFF + i, 0)),
        ],
        out_specs=pl.BlockSpec((8, 128), lambda i: (0, 0)),
        out_shape=jax.ShapeDtypeStruct((8, 128), jnp.float32),
    )(d2, t2, l2)

    a = 0.5 * (jnp.sum(sc_partials[:, 0:16]) + jnp.sum(tc_partials[0, :]))
    pos = jnp.sum(sc_partials[:, 16:32]) + jnp.sum(tc_partials[1, :])
    val = jnp.sum(sc_partials[:, 32:48]) + jnp.sum(tc_partials[2, :])
    return a / (pos + EPS * val)


def kernel(output, target, labels):
    out_flat = jnp.reshape(
        jnp.transpose(jnp.reshape(output, (A // 128, 128, 4)), (0, 2, 1)),
        (-1,))
    tgt_flat = jnp.reshape(
        jnp.transpose(jnp.reshape(target, (A // 128, 128, 4)), (0, 2, 1)),
        (-1,))
    lbl_flat = jnp.reshape(labels, (-1,))
    return _rpn_loss(out_flat, tgt_flat, lbl_flat)
